# baseline (speedup 1.0000x reference)
import functools
import jax
import jax.numpy as jnp
from jax.experimental import pallas as pl
from jax.experimental.pallas import tpu as pltpu
from jax.experimental.pallas import tpu_sc as plsc

B = 4
C = 192
H = 224
W = 224
K = 7
HO = H - K + 1
WO = W - K + 1
NSEL = C // 2
CB = 16


def _pool_min_body(x_ref, o_ref):
    xb = x_ref[...]
    xt = jnp.swapaxes(xb, 1, 2)
    ht = xt[:, 0:WO, :]
    for d in range(1, K):
        ht = ht + xt[:, d:d + WO, :]
    h = jnp.swapaxes(ht, 1, 2)
    v = h[:, 0:HO, :]
    for d in range(1, K):
        v = v + h[:, d:d + HO, :]
    m = jnp.min(v, axis=(1, 2))
    o_ref[0, 0, :] = m


def _pool_min(x):
    xf = x.reshape(B * C, H, W)
    g = (B * C) // CB
    out = pl.pallas_call(
        _pool_min_body,
        grid=(g,),
        in_specs=[pl.BlockSpec((CB, H, W), lambda i: (i, 0, 0))],
        out_specs=pl.BlockSpec((1, 1, CB), lambda i: (i, 0, 0)),
        out_shape=jax.ShapeDtypeStruct((g, 1, CB), jnp.float32),
    )(xf)
    return out.reshape(B, C)


def _select_body(m_ref, sel_ref, per_ref):
    mt = jnp.swapaxes(m_ref[...], 0, 1)
    ii = jax.lax.broadcasted_iota(jnp.int32, (C, C), 0)
    jj = jax.lax.broadcasted_iota(jnp.int32, (C, C), 1)
    ids_col = jax.lax.broadcasted_iota(jnp.int32, (C, NSEL), 0)
    ks = jax.lax.broadcasted_iota(jnp.int32, (C, NSEL), 1)
    for b in range(B):
        m_row = m_ref[b:b + 1, :]
        m_col = mt[:, b:b + 1]
        lt = (m_row < m_col)
        eq = (m_row == m_col)
        cnt = (lt | (eq & (jj < ii))).astype(jnp.int32)
        rank_col = jnp.sum(cnt, axis=1, keepdims=True)
        gt = (m_col < m_row)
        cnt_row = (gt | (eq & (ii < jj))).astype(jnp.int32)
        rank_row = jnp.sum(cnt_row, axis=0, keepdims=True)
        sel_col = rank_col >= NSEL
        sel_row = rank_row >= NSEL
        hit = (rank_col == (ks + NSEL)) & sel_col
        sel_vals = jnp.sum(jnp.where(hit, ids_col, 0), axis=0, keepdims=True)
        sel_ref[b:b + 1, :] = sel_vals
        q_col = jnp.sum((sel_row & (jj < ii)).astype(jnp.int32),
                        axis=1, keepdims=True)
        phit = (q_col == ks) & sel_col
        per_vals = jnp.sum(jnp.where(phit, ids_col, 0), axis=0, keepdims=True)
        per_ref[b:b + 1, :] = per_vals


def _select(mins):
    return pl.pallas_call(
        _select_body,
        in_specs=[
            pl.BlockSpec((B, C), lambda: (0, 0)),
        ],
        out_specs=[
            pl.BlockSpec((B, NSEL), lambda: (0, 0)),
            pl.BlockSpec((B, NSEL), lambda: (0, 0)),
        ],
        out_shape=[
            jax.ShapeDtypeStruct((B, NSEL), jnp.int32),
            jax.ShapeDtypeStruct((B, NSEL), jnp.int32),
        ],
    )(mins)


D = H * W
NW = 32
RPW = (B * NSEL) // NW


def _sc_gather_body(x_ref, gidx_ref, o_ref, idx_v, buf0, buf1, gsem, ssem):
    bufs = (buf0, buf1)
    wid = jax.lax.axis_index("s") * 2 + jax.lax.axis_index("c")
    pltpu.sync_copy(gidx_ref.at[wid], idx_v)
    base = wid * RPW

    def g_start(r):
        return pltpu.async_copy(
            x_ref.at[idx_v.at[pl.ds(8 * r, 1)]],
            bufs[r % 2], gsem)

    def s_start(r):
        return pltpu.async_copy(
            bufs[r % 2],
            o_ref.at[pl.ds(base + r, 1)], ssem)

    g = {0: g_start(0)}
    s = {}
    for r in range(RPW):
        g[r].wait()
        if r >= 1:
            s[r - 1].wait()
        if r + 1 < RPW:
            g[r + 1] = g_start(r + 1)
        s[r] = s_start(r)
    s[RPW - 1].wait()


def _gather(x, sel):
    gidx = (sel + C * jnp.arange(B, dtype=jnp.int32)[:, None])
    gidx2 = jnp.zeros((NW, RPW, 8), jnp.int32).at[:, :, 0].set(
        gidx.reshape(NW, RPW)).reshape(NW, RPW * 8)
    mesh = plsc.VectorSubcoreMesh(core_axis_name="c", subcore_axis_name="s")
    f = pl.kernel(
        _sc_gather_body,
        out_type=jax.ShapeDtypeStruct((B * NSEL, D), jnp.float32),
        mesh=mesh,
        scratch_types=[
            pltpu.VMEM((RPW * 8,), jnp.int32),
            pltpu.VMEM((1, D), jnp.float32),
            pltpu.VMEM((1, D), jnp.float32),
            pltpu.SemaphoreType.DMA,
            pltpu.SemaphoreType.DMA,
        ],
    )
    out2 = f(x.reshape(B * C, D), gidx2)
    return out2.reshape(B, NSEL, H, W)


@jax.jit
def kernel(x):
    mins = _pool_min(x)
    sel, period = _select(mins)
    selected = _gather(x, sel)
    return selected, period


# speedup vs baseline: 1.8059x; 1.8059x over previous
import functools
import jax
import jax.numpy as jnp
from jax.experimental import pallas as pl
from jax.experimental.pallas import tpu as pltpu

B = 4
C = 192
H = 224
W = 224
K = 7
HO = H - K + 1
WO = W - K + 1
NSEL = C // 2
CB = 16


def _pool_min_body(x_ref, o_ref):
    xb = x_ref[...]
    xt = jnp.swapaxes(xb, 1, 2)
    ht = xt[:, 0:WO, :]
    for d in range(1, K):
        ht = ht + xt[:, d:d + WO, :]
    h = jnp.swapaxes(ht, 1, 2)
    v = h[:, 0:HO, :]
    for d in range(1, K):
        v = v + h[:, d:d + HO, :]
    m = jnp.min(v, axis=(1, 2))
    o_ref[0, 0, :] = m


def _pool_min(x):
    xf = x.reshape(B * C, H, W)
    g = (B * C) // CB
    out = pl.pallas_call(
        _pool_min_body,
        grid=(g,),
        in_specs=[pl.BlockSpec((CB, H, W), lambda i: (i, 0, 0))],
        out_specs=pl.BlockSpec((1, 1, CB), lambda i: (i, 0, 0)),
        out_shape=jax.ShapeDtypeStruct((g, 1, CB), jnp.float32),
    )(xf)
    return out.reshape(B, C)


def _select_body(m_ref, sel_ref, per_ref):
    mt = jnp.swapaxes(m_ref[...], 0, 1)
    ii = jax.lax.broadcasted_iota(jnp.int32, (C, C), 0)
    jj = jax.lax.broadcasted_iota(jnp.int32, (C, C), 1)
    ids_col = jax.lax.broadcasted_iota(jnp.int32, (C, NSEL), 0)
    ks = jax.lax.broadcasted_iota(jnp.int32, (C, NSEL), 1)
    for b in range(B):
        m_row = m_ref[b:b + 1, :]
        m_col = mt[:, b:b + 1]
        lt = (m_row < m_col)
        eq = (m_row == m_col)
        cnt = (lt | (eq & (jj < ii))).astype(jnp.int32)
        rank_col = jnp.sum(cnt, axis=1, keepdims=True)
        gt = (m_col < m_row)
        cnt_row = (gt | (eq & (ii < jj))).astype(jnp.int32)
        rank_row = jnp.sum(cnt_row, axis=0, keepdims=True)
        sel_col = rank_col >= NSEL
        sel_row = rank_row >= NSEL
        hit = (rank_col == (ks + NSEL)) & sel_col
        sel_vals = jnp.sum(jnp.where(hit, ids_col, 0), axis=0, keepdims=True)
        sel_ref[b:b + 1, :] = sel_vals
        q_col = jnp.sum((sel_row & (jj < ii)).astype(jnp.int32),
                        axis=1, keepdims=True)
        phit = (q_col == ks) & sel_col
        per_vals = jnp.sum(jnp.where(phit, ids_col, 0), axis=0, keepdims=True)
        per_ref[b:b + 1, :] = per_vals


def _select(mins):
    return pl.pallas_call(
        _select_body,
        in_specs=[
            pl.BlockSpec((B, C), lambda: (0, 0)),
        ],
        out_specs=[
            pl.BlockSpec((B, NSEL), lambda: (0, 0)),
            pl.BlockSpec((B, NSEL), lambda: (0, 0)),
        ],
        out_shape=[
            jax.ShapeDtypeStruct((B, NSEL), jnp.int32),
            jax.ShapeDtypeStruct((B, NSEL), jnp.int32),
        ],
    )(mins)


GG = 16


def _gather_body(sel_ref, *refs):
    o_ref = refs[-1]
    for g in range(GG):
        o_ref[0, g] = refs[g][0, 0]


def _gather(x, sel):
    sel_flat = sel.reshape(B * NSEL)
    in_specs = [
        pl.BlockSpec((1, 1, H, W),
                     functools.partial(
                         lambda g, i, sel_ref:
                         ((i * GG) // NSEL, sel_ref[i * GG + g], 0, 0), g))
        for g in range(GG)
    ]
    grid_spec = pltpu.PrefetchScalarGridSpec(
        num_scalar_prefetch=1,
        grid=((B * NSEL) // GG,),
        in_specs=in_specs,
        out_specs=pl.BlockSpec((1, GG, H, W),
                               lambda i, sel_ref: ((i * GG) // NSEL,
                                                   i % (NSEL // GG), 0, 0)),
    )
    return pl.pallas_call(
        _gather_body,
        grid_spec=grid_spec,
        out_shape=jax.ShapeDtypeStruct((B, NSEL, H, W), jnp.float32),
    )(sel_flat, *([x] * GG))


@jax.jit
def kernel(x):
    mins = _pool_min(x)
    sel, period = _select(mins)
    selected = _gather(x, sel)
    return selected, period
